# baseline (speedup 1.0000x reference)
import functools

import jax
import jax.numpy as jnp
from jax import lax
from jax.experimental import pallas as pl
from jax.experimental.pallas import tpu as pltpu
from jax.experimental.pallas import tpu_sc as plsc

_NUM_TOKENS = 32768
_NUM_EXPERTS = 64
_TOP_K = 2
_BLOCK = 2048

_NC = 2
_NS = 16
_NW = _NC * _NS
_CHUNK = _NUM_TOKENS // _NW


def _gate_body(x_ref, w_ref, b_ref, sel_ref, wt_ref, aux_ref, cnt_ref, prob_ref):
    i = pl.program_id(0)
    nsteps = pl.num_programs(0)
    e = _NUM_EXPERTS

    logits = jnp.dot(x_ref[...], w_ref[...], preferred_element_type=jnp.float32)
    logits = logits + b_ref[...]

    m1 = jnp.max(logits, axis=1, keepdims=True)
    sel1 = logits == m1
    sel1f = jnp.where(sel1, 1.0, 0.0)

    masked = jnp.where(sel1, -jnp.inf, logits)
    m2 = jnp.max(masked, axis=1, keepdims=True)
    mask2f = jnp.where(logits >= m2, 1.0, 0.0)

    iota_mat = jax.lax.broadcasted_iota(jnp.int32, (e, e), 0).astype(jnp.float32)
    i1f = jnp.dot(sel1f, iota_mat, preferred_element_type=jnp.float32)
    i12f = jnp.dot(mask2f, iota_mat, preferred_element_type=jnp.float32)

    p_un = jnp.exp(logits - m1)

    ones_e = jnp.full((e, e), 1.0, dtype=jnp.float32)
    s_full = jnp.dot(p_un, ones_e, preferred_element_type=jnp.float32)

    e2 = jnp.exp(m2 - m1)
    w1 = 1.0 / (1.0 + e2)
    w1c = w1[:, 0:1]
    wt_ref[...] = jnp.concatenate([w1c, 1.0 - w1c], axis=1).T

    i1c = i1f[:, 0:1]
    i2c = i12f[:, 0:1] - i1c
    sel_pair = jnp.concatenate(
        [i1c.astype(jnp.int32), i2c.astype(jnp.int32)], axis=1)
    sel_ref[...] = sel_pair.T

    q = p_un / s_full

    @pl.when(i == 0)
    def _():
        cnt_ref[...] = jnp.zeros_like(cnt_ref)
        prob_ref[...] = jnp.zeros_like(prob_ref)

    ones_rows = jnp.full((8, _BLOCK), 1.0, dtype=jnp.float32)
    cnt_ref[...] += jnp.dot(ones_rows, mask2f, preferred_element_type=jnp.float32)
    prob_ref[...] += jnp.dot(ones_rows, q, preferred_element_type=jnp.float32)

    @pl.when(i == nsteps - 1)
    def _():
        scale = _NUM_EXPERTS / (_NUM_TOKENS * _TOP_K * _NUM_TOKENS * 64.0)
        cnt1 = jnp.sum(cnt_ref[...], axis=0, keepdims=True)
        prob1 = jnp.sum(prob_ref[...], axis=0, keepdims=True)
        aux_ref[...] = scale * jnp.sum(cnt1 * prob1, keepdims=True)


def _sc_scatter_body(sel_hbm, w_hbm, out_hbm, idx_v, w_v, rows_v):
    wid = lax.axis_index("s") * _NC + lax.axis_index("c")
    base = wid * _CHUNK
    e = _NUM_EXPERTS

    pltpu.sync_copy(sel_hbm.at[0, pl.ds(base, _CHUNK)], idx_v.at[pl.ds(0, _CHUNK)])
    pltpu.sync_copy(sel_hbm.at[1, pl.ds(base, _CHUNK)],
                    idx_v.at[pl.ds(_CHUNK, _CHUNK)])
    pltpu.sync_copy(w_hbm.at[0, pl.ds(base, _CHUNK)], w_v.at[pl.ds(0, _CHUNK)])
    pltpu.sync_copy(w_hbm.at[1, pl.ds(base, _CHUNK)],
                    w_v.at[pl.ds(_CHUNK, _CHUNK)])

    zeros16 = jnp.zeros((16,), jnp.float32)
    lane = lax.broadcasted_iota(jnp.int32, (16,), 0)
    half = _CHUNK // 2

    for h in range(2):
        def _zero(j, carry):
            rows_v[j, pl.ds(0, 16)] = zeros16
            rows_v[j, pl.ds(16, 16)] = zeros16
            rows_v[j, pl.ds(32, 16)] = zeros16
            rows_v[j, pl.ds(48, 16)] = zeros16
            return carry

        lax.fori_loop(0, half, _zero, 0)

        for g in range(half // 16):
            toks = lane + g * 16
            off = h * half + g * 16
            i1v = idx_v[pl.ds(off, 16)]
            i2v = idx_v[pl.ds(_CHUNK + off, 16)]
            w1v = w_v[pl.ds(off, 16)]
            w2v = w_v[pl.ds(_CHUNK + off, 16)]
            plsc.store_scatter(rows_v, [toks, i1v], w1v)
            plsc.store_scatter(rows_v, [toks, i2v], w2v)

        pltpu.sync_copy(rows_v, out_hbm.at[pl.ds(base + h * half, half), :])


def _sc_scatter(sel_t, w_t):
    mesh = plsc.VectorSubcoreMesh(core_axis_name="c", subcore_axis_name="s")
    f = functools.partial(
        pl.kernel,
        mesh=mesh,
        out_type=jax.ShapeDtypeStruct((_NUM_TOKENS, _NUM_EXPERTS), jnp.float32),
        scratch_types=[
            pltpu.VMEM((2 * _CHUNK,), jnp.int32),
            pltpu.VMEM((2 * _CHUNK,), jnp.float32),
            pltpu.VMEM((_CHUNK // 2, _NUM_EXPERTS), jnp.float32),
        ],
        compiler_params=pltpu.CompilerParams(needs_layout_passes=False),
    )(_sc_scatter_body)
    return f(sel_t, w_t)


def kernel(x, W, b):
    n, d = x.shape
    e = W.shape[1]
    grid = (n // _BLOCK,)
    sel_t, w_t, aux = pl.pallas_call(
        _gate_body,
        grid=grid,
        in_specs=[
            pl.BlockSpec((_BLOCK, d), lambda i: (i, 0)),
            pl.BlockSpec((d, e), lambda i: (0, 0)),
            pl.BlockSpec((1, e), lambda i: (0, 0)),
        ],
        out_specs=[
            pl.BlockSpec((_TOP_K, _BLOCK), lambda i: (0, i)),
            pl.BlockSpec((_TOP_K, _BLOCK), lambda i: (0, i)),
            pl.BlockSpec((1, 1), lambda i: (0, 0)),
        ],
        out_shape=[
            jax.ShapeDtypeStruct((_TOP_K, n), jnp.int32),
            jax.ShapeDtypeStruct((_TOP_K, n), jnp.float32),
            jax.ShapeDtypeStruct((1, 1), jnp.float32),
        ],
        scratch_shapes=[
            pltpu.VMEM((8, e), jnp.float32),
            pltpu.VMEM((8, e), jnp.float32),
        ],
    )(x, W, b.reshape(1, e))
    gw = _sc_scatter(sel_t, w_t)
    return gw, sel_t.T, aux[0, 0]


# speedup vs baseline: 1.0130x; 1.0130x over previous
import functools

import jax
import jax.numpy as jnp
from jax import lax
from jax.experimental import pallas as pl
from jax.experimental.pallas import tpu as pltpu
from jax.experimental.pallas import tpu_sc as plsc

_NUM_TOKENS = 32768
_NUM_EXPERTS = 64
_TOP_K = 2
_BLOCK = 4096

_NC = 2
_NS = 16
_NW = _NC * _NS
_CHUNK = _NUM_TOKENS // _NW


def _gate_body(x_ref, w_ref, b_ref, sel_ref, wt_ref, aux_ref, cnt_ref, prob_ref):
    i = pl.program_id(0)
    nsteps = pl.num_programs(0)
    e = _NUM_EXPERTS

    logits = jnp.dot(x_ref[...], w_ref[...], preferred_element_type=jnp.float32)
    logits = logits + b_ref[...]

    m1 = jnp.max(logits, axis=1, keepdims=True)
    sel1 = logits == m1
    sel1f = jnp.where(sel1, 1.0, 0.0)

    masked = jnp.where(sel1, -jnp.inf, logits)
    m2 = jnp.max(masked, axis=1, keepdims=True)
    mask2f = jnp.where(logits >= m2, 1.0, 0.0)

    iota_mat = jax.lax.broadcasted_iota(jnp.int32, (e, e), 0).astype(jnp.float32)
    i1f = jnp.dot(sel1f, iota_mat, preferred_element_type=jnp.float32)
    i12f = jnp.dot(mask2f, iota_mat, preferred_element_type=jnp.float32)

    p_un = jnp.exp(logits - m1)

    ones_e = jnp.full((e, e), 1.0, dtype=jnp.float32)
    s_full = jnp.dot(p_un, ones_e, preferred_element_type=jnp.float32)

    e2 = jnp.exp(m2 - m1)
    w1 = 1.0 / (1.0 + e2)
    w1c = w1[:, 0:1]
    wt_ref[...] = jnp.concatenate([w1c, 1.0 - w1c], axis=1).T

    i1c = i1f[:, 0:1]
    i2c = i12f[:, 0:1] - i1c
    sel_pair = jnp.concatenate(
        [i1c.astype(jnp.int32), i2c.astype(jnp.int32)], axis=1)
    sel_ref[...] = sel_pair.T

    q = p_un / s_full

    @pl.when(i == 0)
    def _():
        cnt_ref[...] = jnp.zeros_like(cnt_ref)
        prob_ref[...] = jnp.zeros_like(prob_ref)

    ones_rows = jnp.full((8, _BLOCK), 1.0, dtype=jnp.float32)
    cnt_ref[...] += jnp.dot(ones_rows, mask2f, preferred_element_type=jnp.float32)
    prob_ref[...] += jnp.dot(ones_rows, q, preferred_element_type=jnp.float32)

    @pl.when(i == nsteps - 1)
    def _():
        scale = _NUM_EXPERTS / (_NUM_TOKENS * _TOP_K * _NUM_TOKENS * 64.0)
        cnt1 = jnp.sum(cnt_ref[...], axis=0, keepdims=True)
        prob1 = jnp.sum(prob_ref[...], axis=0, keepdims=True)
        aux_ref[...] = scale * jnp.sum(cnt1 * prob1, keepdims=True)


def _sc_scatter_body(sel_hbm, w_hbm, out_hbm, idx_v, w_v, rows_v):
    wid = lax.axis_index("s") * _NC + lax.axis_index("c")
    base = wid * _CHUNK
    e = _NUM_EXPERTS

    pltpu.sync_copy(sel_hbm.at[0, pl.ds(base, _CHUNK)], idx_v.at[pl.ds(0, _CHUNK)])
    pltpu.sync_copy(sel_hbm.at[1, pl.ds(base, _CHUNK)],
                    idx_v.at[pl.ds(_CHUNK, _CHUNK)])
    pltpu.sync_copy(w_hbm.at[0, pl.ds(base, _CHUNK)], w_v.at[pl.ds(0, _CHUNK)])
    pltpu.sync_copy(w_hbm.at[1, pl.ds(base, _CHUNK)],
                    w_v.at[pl.ds(_CHUNK, _CHUNK)])

    zeros16 = jnp.zeros((16,), jnp.float32)
    lane = lax.broadcasted_iota(jnp.int32, (16,), 0)
    half = _CHUNK // 2

    for h in range(2):
        def _zero(j, carry):
            rows_v[j, pl.ds(0, 16)] = zeros16
            rows_v[j, pl.ds(16, 16)] = zeros16
            rows_v[j, pl.ds(32, 16)] = zeros16
            rows_v[j, pl.ds(48, 16)] = zeros16
            return carry

        lax.fori_loop(0, half, _zero, 0)

        for g in range(half // 16):
            toks = lane + g * 16
            off = h * half + g * 16
            i1v = idx_v[pl.ds(off, 16)]
            i2v = idx_v[pl.ds(_CHUNK + off, 16)]
            w1v = w_v[pl.ds(off, 16)]
            w2v = w_v[pl.ds(_CHUNK + off, 16)]
            plsc.store_scatter(rows_v, [toks, i1v], w1v)
            plsc.store_scatter(rows_v, [toks, i2v], w2v)

        pltpu.sync_copy(rows_v, out_hbm.at[pl.ds(base + h * half, half), :])


def _sc_scatter(sel_t, w_t):
    mesh = plsc.VectorSubcoreMesh(core_axis_name="c", subcore_axis_name="s")
    f = functools.partial(
        pl.kernel,
        mesh=mesh,
        out_type=jax.ShapeDtypeStruct((_NUM_TOKENS, _NUM_EXPERTS), jnp.float32),
        scratch_types=[
            pltpu.VMEM((2 * _CHUNK,), jnp.int32),
            pltpu.VMEM((2 * _CHUNK,), jnp.float32),
            pltpu.VMEM((_CHUNK // 2, _NUM_EXPERTS), jnp.float32),
        ],
        compiler_params=pltpu.CompilerParams(needs_layout_passes=False),
    )(_sc_scatter_body)
    return f(sel_t, w_t)


def kernel(x, W, b):
    n, d = x.shape
    e = W.shape[1]
    grid = (n // _BLOCK,)
    sel_t, w_t, aux = pl.pallas_call(
        _gate_body,
        grid=grid,
        in_specs=[
            pl.BlockSpec((_BLOCK, d), lambda i: (i, 0)),
            pl.BlockSpec((d, e), lambda i: (0, 0)),
            pl.BlockSpec((1, e), lambda i: (0, 0)),
        ],
        out_specs=[
            pl.BlockSpec((_TOP_K, _BLOCK), lambda i: (0, i)),
            pl.BlockSpec((_TOP_K, _BLOCK), lambda i: (0, i)),
            pl.BlockSpec((1, 1), lambda i: (0, 0)),
        ],
        out_shape=[
            jax.ShapeDtypeStruct((_TOP_K, n), jnp.int32),
            jax.ShapeDtypeStruct((_TOP_K, n), jnp.float32),
            jax.ShapeDtypeStruct((1, 1), jnp.float32),
        ],
        scratch_shapes=[
            pltpu.VMEM((8, e), jnp.float32),
            pltpu.VMEM((8, e), jnp.float32),
        ],
    )(x, W, b.reshape(1, e))
    gw = _sc_scatter(sel_t, w_t)
    return gw, sel_t.T, aux[0, 0]
